# baseline (speedup 1.0000x reference)
import functools

import jax
import jax.numpy as jnp
from jax import lax
from jax.experimental import pallas as pl
from jax.experimental.pallas import tpu as pltpu
from jax.experimental.pallas import tpu_sc as plsc

D = 256
T = 512
TROWS = 513
Q_ROWS = 1024
NC = 2
NS = 16
NPH = 4
CHUNK = Q_ROWS // NS
ROWS_PER_TILE_PER_PH = 4
BLK = 72
LANES = 16

_mesh = plsc.VectorSubcoreMesh(core_axis_name="c", subcore_axis_name="s")


@functools.partial(
    pl.kernel,
    out_type=jax.ShapeDtypeStruct((T, T, D), jnp.float32),
    mesh=_mesh,
    scratch_types=[
        pltpu.VMEM_SHARED((NPH, Q_ROWS, D), jnp.float32),
        pltpu.VMEM((BLK, D), jnp.float32),
        pltpu.VMEM((LANES,), jnp.int32),
        pltpu.VMEM((LANES, D), jnp.float32),
        pltpu.VMEM((CHUNK, D), jnp.float32),
        pltpu.VMEM((CHUNK, D), jnp.float32),
        pltpu.SemaphoreType.DMA,
        pltpu.SemaphoreType.DMA,
        pltpu.SemaphoreType.DMA,
        pltpu.SemaphoreType.DMA,
    ],
)
def _rpe_sc(table_hbm, out_hbm, q_sh, blk_v, idx_l, last_v, ra, rb,
            lsem, pa, pb, sem):
    c = lax.axis_index("c")
    s = lax.axis_index("s")
    rows = [ra, rb]
    psems = [pa, pb]

    base = pl.multiple_of(jnp.clip(CHUNK * s - 256, 0, 440), 8)
    lane = lax.iota(jnp.int32, LANES)
    idx_l[pl.ds(0, LANES)] = lane * 0 + (TROWS - 1)
    ld_b = pltpu.async_copy(table_hbm.at[pl.ds(base, BLK), :], blk_v, lsem)
    ld_b.wait()
    ld_l = pltpu.async_copy(table_hbm.at[idx_l], last_v, lsem)
    ld_l.wait()

    last_regs = [last_v[0, pl.ds(LANES * u, LANES)] for u in range(D // LANES)]

    def build(t, b):
        phi = NPH * c + t
        u0 = CHUNK * s + phi - 256

        def body(rr, carry):
            for half in range(2):
                r = 2 * rr + half
                v = u0 + r
                vloc = jnp.clip(v, 0, TROWS - 1) - base
                use_last = vloc >= BLK
                vl = jnp.minimum(vloc, BLK - 1)
                for u in range(D // LANES):
                    x = blk_v[vl, pl.ds(LANES * u, LANES)]
                    rows[b][r, pl.ds(LANES * u, LANES)] = jnp.where(
                        use_last, last_regs[u], x)
            return carry

        lax.fori_loop(0, CHUNK // 2, body, 0)

    def publish(t, b):
        return pltpu.async_copy(
            rows[b], q_sh.at[t, pl.ds(CHUNK * s, CHUNK), :], psems[b])

    outs = []

    def fire(t, kks):
        phi = NPH * c + t
        rem = (8 - phi) % 8
        off = jnp.where(phi > 0, 1, 0)
        for kk in kks:
            k = ROWS_PER_TILE_PER_PH * s + kk
            i = 8 * k + rem
            a = 64 - k - off
            outs.append(pltpu.async_copy(
                q_sh.at[t, pl.ds(8 * a, T), :],
                out_hbm.at[i],
                sem))

    build(0, 0)
    pub = publish(0, 0)
    pub.wait()
    plsc.subcore_barrier()
    for t in range(1, NPH):
        fire(t - 1, [0])
        build(t, t % 2)
        pub = publish(t, t % 2)
        fire(t - 1, [1, 2, 3])
        pub.wait()
        plsc.subcore_barrier()
    fire(NPH - 1, [0, 1, 2, 3])
    for cp in outs:
        cp.wait()


def kernel(length, table):
    del length
    return _rpe_sc(table)


# speedup vs baseline: 1.0129x; 1.0129x over previous
import functools

import jax
import jax.numpy as jnp
from jax import lax
from jax.experimental import pallas as pl
from jax.experimental.pallas import tpu as pltpu
from jax.experimental.pallas import tpu_sc as plsc

D = 256
T = 512
TROWS = 513
Q_ROWS = 1024
NC = 2
NS = 16
NPH = 4
CHUNK = Q_ROWS // NS
ROWS_PER_TILE_PER_PH = 4
BLK = 72
LANES = 16

_mesh = plsc.VectorSubcoreMesh(core_axis_name="c", subcore_axis_name="s")


@functools.partial(
    pl.kernel,
    out_type=jax.ShapeDtypeStruct((T, T, D), jnp.float32),
    mesh=_mesh,
    scratch_types=[
        pltpu.VMEM_SHARED((NPH, Q_ROWS, D), jnp.float32),
        pltpu.VMEM((BLK, D), jnp.float32),
        pltpu.VMEM((LANES,), jnp.int32),
        pltpu.VMEM((LANES, D), jnp.float32),
        pltpu.VMEM((CHUNK, D), jnp.float32),
        pltpu.VMEM((CHUNK, D), jnp.float32),
        pltpu.SemaphoreType.DMA,
        pltpu.SemaphoreType.DMA,
        pltpu.SemaphoreType.DMA,
        pltpu.SemaphoreType.DMA,
    ],
)
def _rpe_sc(table_hbm, out_hbm, q_sh, blk_v, idx_l, last_v, ra, rb,
            lsem, pa, pb, sem):
    c = lax.axis_index("c")
    s = lax.axis_index("s")
    rows = [ra, rb]
    psems = [pa, pb]

    base = pl.multiple_of(jnp.clip(CHUNK * s - 256, 0, 440), 8)
    lane = lax.iota(jnp.int32, LANES)
    idx_l[pl.ds(0, LANES)] = lane * 0 + (TROWS - 1)
    ld_b = pltpu.async_copy(table_hbm.at[pl.ds(base, BLK), :], blk_v, lsem)
    ld_b.wait()
    ld_l = pltpu.async_copy(table_hbm.at[idx_l], last_v, lsem)
    ld_l.wait()

    last_regs = [last_v[0, pl.ds(LANES * u, LANES)] for u in range(D // LANES)]

    def build(t, b):
        phi = NPH * c + t
        u0 = CHUNK * s + phi - 256

        def body(r, carry):
            v = u0 + r
            vloc = jnp.clip(v, 0, TROWS - 1) - base
            use_last = vloc >= BLK
            vl = jnp.minimum(vloc, BLK - 1)
            for u in range(D // LANES):
                x = blk_v[vl, pl.ds(LANES * u, LANES)]
                rows[b][r, pl.ds(LANES * u, LANES)] = jnp.where(
                    use_last, last_regs[u], x)
            return carry

        lax.fori_loop(0, CHUNK, body, 0)

    def publish(t, b):
        return pltpu.async_copy(
            rows[b], q_sh.at[t, pl.ds(CHUNK * s, CHUNK), :], psems[b])

    outs = []

    def fire(t):
        phi = NPH * c + t
        rem = (8 - phi) % 8
        off = jnp.where(phi > 0, 1, 0)
        for kk in range(ROWS_PER_TILE_PER_PH):
            k = ROWS_PER_TILE_PER_PH * s + kk
            i = 8 * k + rem
            a = 64 - k - off
            outs.append(pltpu.async_copy(
                q_sh.at[t, pl.ds(8 * a, T), :],
                out_hbm.at[i],
                sem))

    for t in range(NPH):
        b = t % 2
        build(t, b)
        pub = publish(t, b)
        pub.wait()
        plsc.subcore_barrier()
        fire(t)
    for cp in outs:
        cp.wait()


def kernel(length, table):
    del length
    return _rpe_sc(table)


# speedup vs baseline: 1.0343x; 1.0211x over previous
import functools

import jax
import jax.numpy as jnp
from jax import lax
from jax.experimental import pallas as pl
from jax.experimental.pallas import tpu as pltpu
from jax.experimental.pallas import tpu_sc as plsc

D = 256
T = 512
TROWS = 513
Q_ROWS = 1024
NC = 2
NS = 16
NPH = 4
CHUNK = Q_ROWS // NS
ROWS_PER_TILE_PER_PH = 4
BLK = 72
LANES = 16

_mesh = plsc.VectorSubcoreMesh(core_axis_name="c", subcore_axis_name="s")


@functools.partial(
    pl.kernel,
    out_type=jax.ShapeDtypeStruct((T, T, D), jnp.float32),
    mesh=_mesh,
    scratch_types=[
        pltpu.VMEM_SHARED((NPH, Q_ROWS, D), jnp.float32),
        pltpu.VMEM((BLK, D), jnp.float32),
        pltpu.VMEM((LANES,), jnp.int32),
        pltpu.VMEM((LANES, D), jnp.float32),
        pltpu.VMEM((CHUNK, D), jnp.float32),
        pltpu.VMEM((CHUNK, D), jnp.float32),
        pltpu.SemaphoreType.DMA,
        pltpu.SemaphoreType.DMA,
        pltpu.SemaphoreType.DMA,
        pltpu.SemaphoreType.DMA,
    ],
)
def _rpe_sc(table_hbm, out_hbm, q_sh, blk_v, idx_l, last_v, ra, rb,
            lsem, pa, pb, sem):
    c = lax.axis_index("c")
    s = lax.axis_index("s")
    rows = [ra, rb]
    psems = [pa, pb]

    base = pl.multiple_of(jnp.clip(CHUNK * s - 256, 0, 440), 8)
    lane = lax.iota(jnp.int32, LANES)
    idx_l[pl.ds(0, LANES)] = lane * 0 + (TROWS - 1)
    ld_b = pltpu.async_copy(table_hbm.at[pl.ds(base, BLK), :], blk_v, lsem)
    ld_b.wait()
    ld_l = pltpu.async_copy(table_hbm.at[idx_l], last_v, lsem)
    ld_l.wait()

    last_regs = [last_v[0, pl.ds(LANES * u, LANES)] for u in range(D // LANES)]

    def build(t, b):
        phi = NPH * c + t
        u0 = CHUNK * s + phi - 256

        @functools.partial(plsc.parallel_loop, 0, CHUNK, unroll=4)
        def _(r):
            v = u0 + r
            vloc = jnp.clip(v, 0, TROWS - 1) - base
            use_last = vloc >= BLK
            vl = jnp.minimum(vloc, BLK - 1)
            for u in range(D // LANES):
                x = blk_v[vl, pl.ds(LANES * u, LANES)]
                rows[b][r, pl.ds(LANES * u, LANES)] = jnp.where(
                    use_last, last_regs[u], x)

    def publish(t, b):
        return pltpu.async_copy(
            rows[b], q_sh.at[t, pl.ds(CHUNK * s, CHUNK), :], psems[b])

    outs = []

    def fire(t):
        phi = NPH * c + t
        rem = (8 - phi) % 8
        off = jnp.where(phi > 0, 1, 0)
        for kk in range(ROWS_PER_TILE_PER_PH):
            k = ROWS_PER_TILE_PER_PH * s + kk
            i = 8 * k + rem
            a = 64 - k - off
            outs.append(pltpu.async_copy(
                q_sh.at[t, pl.ds(8 * a, T), :],
                out_hbm.at[i],
                sem))

    for t in range(NPH):
        b = t % 2
        build(t, b)
        pub = publish(t, b)
        pub.wait()
        plsc.subcore_barrier()
        fire(t)
    for cp in outs:
        cp.wait()


def kernel(length, table):
    del length
    return _rpe_sc(table)
